# Initial kernel scaffold; baseline (speedup 1.0000x reference)
import jax
import jax.numpy as jnp
from jax.experimental import pallas as pl


def kernel(x, params):
    raise NotImplementedError("write your pallas kernel here")


# speedup vs baseline: 9.3450x; 9.3450x over previous
import functools

import jax
import jax.numpy as jnp
from jax import lax
from jax.experimental import pallas as pl
from jax.experimental.pallas import tpu as pltpu
from jax.experimental.pallas import tpu_sc as plsc

KNN = 20
EPS = 1e-5
B = 4
N = 2048
M = B * N
RT = 256
F32 = jnp.float32


def _topk_conv_body(xc_full_ref, xt_full_ref, xt_tile_ref, w_ref, idx_ref, y_ref):
    b = pl.program_id(0)
    xc_full = xc_full_ref[0]
    xt_full = xt_full_ref[0]
    xt_tile = xt_tile_ref[0]
    nt = (((1,), (1,)), ((), ()))
    inner = lax.dot_general(xt_tile, xc_full, (((1,), (0,)), ((), ())),
                            preferred_element_type=F32)
    xx_full = jnp.sum(xc_full * xc_full, axis=0, keepdims=True)
    xx_tile = jnp.sum(xt_tile * xt_tile, axis=1, keepdims=True)
    s = (2.0 * inner - xx_tile) - xx_full
    col = lax.broadcasted_iota(jnp.int32, s.shape, 1)
    colk = lax.broadcasted_iota(jnp.int32, (RT, KNN), 1)
    idx = jnp.zeros((RT, KNN), jnp.int32)
    for j in range(KNN):
        m = jnp.max(s, axis=1, keepdims=True)
        cand = jnp.where(s == m, col, N)
        am = jnp.min(cand, axis=1, keepdims=True)
        idx = jnp.where(colk == j, am, idx)
        s = jnp.where(col == am, -jnp.inf, s)
    idx_ref[0] = idx + b * N
    y_ref[0] = lax.dot_general(xt_tile, w_ref[...], nt, preferred_element_type=F32)


def _topk_conv(xT, W):
    _, Nn, Cp = xT.shape
    O = W.shape[0]
    xC = jnp.transpose(xT, (0, 2, 1))
    return pl.pallas_call(
        _topk_conv_body,
        grid=(B, Nn // RT),
        in_specs=[
            pl.BlockSpec((1, Cp, Nn), lambda b, i: (b, 0, 0)),
            pl.BlockSpec((1, Nn, Cp), lambda b, i: (b, 0, 0)),
            pl.BlockSpec((1, RT, Cp), lambda b, i: (b, i, 0)),
            pl.BlockSpec((O, Cp), lambda b, i: (0, 0)),
        ],
        out_specs=[
            pl.BlockSpec((1, RT, KNN), lambda b, i: (b, i, 0)),
            pl.BlockSpec((1, RT, O), lambda b, i: (b, i, 0)),
        ],
        out_shape=[
            jax.ShapeDtypeStruct((B, Nn, KNN), jnp.int32),
            jax.ShapeDtypeStruct((B, Nn, O), F32),
        ],
    )(xC, xT, xT, W)


_NW = 32
_RW = M // _NW
_CH = 64
_NCH = _RW // _CH
_G = _CH * KNN // 128


def _sc_gather_max(yflat, idx2d):
    mesh = plsc.VectorSubcoreMesh(core_axis_name="c", subcore_axis_name="s")

    @functools.partial(
        pl.kernel,
        mesh=mesh,
        compiler_params=pltpu.CompilerParams(use_tc_tiling_on_sc=False),
        out_type=[
            jax.ShapeDtypeStruct((M, 64), F32),
            jax.ShapeDtypeStruct((_NW, 128), F32),
        ],
        scratch_types=[
            pltpu.VMEM((_RW * KNN // 128, 128), jnp.int32),
            pltpu.VMEM((_G, 128, 64), F32),
            pltpu.VMEM((_CH, 64), F32),
            pltpu.VMEM((128,), F32),
            pltpu.SemaphoreType.DMA,
        ],
    )
    def k(y_hbm, idx_hbm, maxed_hbm, part_hbm, idx_v, rows_v, out_v, acc_v, sem):
        wid = lax.axis_index("s") * 2 + lax.axis_index("c")
        zero = jnp.zeros((16,), F32)
        nrows = _RW * KNN // 128
        pltpu.sync_copy(idx_hbm.at[pl.ds(wid * nrows, nrows)], idx_v)

        def chunk_body(ci, carry):
            row0 = wid * _RW + ci * _CH
            cps = [pltpu.async_copy(y_hbm.at[idx_v.at[ci * _G + g]], rows_v.at[g], sem)
                   for g in range(_G)]
            for cp in cps:
                cp.wait()

            def t_body(t, c2):
                accs = list(c2[0:8])
                comp = list(c2[8:16])
                flat0 = t * KNN
                ms = [None] * 4
                for j in range(KNN):
                    g = (flat0 + j) // 128
                    r = (flat0 + j) % 128
                    vs = [rows_v[g, r, pl.ds(c * 16, 16)] for c in range(4)]
                    for c in range(4):
                        v = vs[c]
                        ms[c] = v if j == 0 else jnp.maximum(ms[c], v)
                        for a, val in ((c, v), (4 + c, v * v)):
                            y = val - comp[a]
                            tt = accs[a] + y
                            comp[a] = (tt - accs[a]) - y
                            accs[a] = tt
                for c in range(4):
                    out_v[t, pl.ds(c * 16, 16)] = ms[c]
                return tuple(accs + comp)

            carry = lax.fori_loop(0, _CH, t_body, carry)
            pltpu.sync_copy(out_v, maxed_hbm.at[pl.ds(row0, _CH)])
            return carry

        carry = lax.fori_loop(0, _NCH, chunk_body, (zero,) * 16)
        for c in range(8):
            acc_v[pl.ds(c * 16, 16)] = carry[c]
        pltpu.sync_copy(acc_v, part_hbm.at[wid])

    return k(yflat, idx2d)


def _finalize_body(maxed_ref, part_ref, gb_ref, out_ref):
    tot = jnp.sum(part_ref[...], axis=0, keepdims=True)
    cnt = float(B * KNN * N)
    mean = tot[:, 0:64] / cnt
    var = tot[:, 64:128] / cnt - mean * mean
    xh = (maxed_ref[...] - mean) / jnp.sqrt(var + EPS)
    out_ref[...] = jnp.maximum(xh * gb_ref[0:1] + gb_ref[1:2], 0.0)


def _finalize(maxed, parts, gb):
    return pl.pallas_call(
        _finalize_body,
        grid=(M // RT,),
        in_specs=[
            pl.BlockSpec((RT, 64), lambda i: (i, 0)),
            pl.BlockSpec((_NW, 128), lambda i: (0, 0)),
            pl.BlockSpec((8, 64), lambda i: (0, 0)),
        ],
        out_specs=pl.BlockSpec((RT, 64), lambda i: (i, 0)),
        out_shape=jax.ShapeDtypeStruct((M, 64), F32),
    )(maxed, parts, gb)


def _edge_layer(xT, Wp, gb):
    idx, yT = _topk_conv(xT, Wp)
    maxed, parts = _sc_gather_max(yT.reshape(M, 64), idx.reshape(-1, 128))
    return _finalize(maxed, parts, gb).reshape(B, N, 64)


def _m1_body(h_ref, w5_ref, y5_ref, s5_ref, m5_ref):
    b = pl.program_id(0)
    i = pl.program_id(1)
    nt = (((1,), (1,)), ((), ()))
    y5 = lax.dot_general(h_ref[0], w5_ref[...], nt, preferred_element_type=F32)
    y5_ref[0] = y5
    tmax = jnp.max(y5, axis=0, keepdims=True)

    @pl.when(jnp.logical_and(b == 0, i == 0))
    def _():
        s5_ref[...] = jnp.zeros_like(s5_ref)
        m5_ref[...] = jnp.full((B, 1024), -jnp.inf, F32)

    s5_ref[0:1, :] += jnp.sum(y5, axis=0, keepdims=True)
    row = lax.broadcasted_iota(jnp.int32, (B, 1024), 0)
    m5_ref[...] = jnp.where(row == b, jnp.maximum(m5_ref[...], tmax), m5_ref[...])


def _m1(h4, W5):
    return pl.pallas_call(
        _m1_body,
        grid=(B, N // RT),
        in_specs=[
            pl.BlockSpec((1, RT, 64), lambda b, i: (b, i, 0)),
            pl.BlockSpec((1024, 64), lambda b, i: (0, 0)),
        ],
        out_specs=[
            pl.BlockSpec((1, RT, 1024), lambda b, i: (b, i, 0)),
            pl.BlockSpec((8, 1024), lambda b, i: (0, 0)),
            pl.BlockSpec((B, 1024), lambda b, i: (0, 0)),
        ],
        out_shape=[
            jax.ShapeDtypeStruct((B, N, 1024), F32),
            jax.ShapeDtypeStruct((8, 1024), F32),
            jax.ShapeDtypeStruct((B, 1024), F32),
        ],
    )(h4, W5)


def _csq_body(y_ref, s_ref, q_ref):
    b = pl.program_id(0)
    i = pl.program_id(1)
    mean = s_ref[0:1, :] / float(M)
    dlt = y_ref[0] - mean

    @pl.when(jnp.logical_and(b == 0, i == 0))
    def _():
        q_ref[...] = jnp.zeros_like(q_ref)

    q_ref[0:1, :] += jnp.sum(dlt * dlt, axis=0, keepdims=True)


def _csq(y, s):
    W = y.shape[2]
    return pl.pallas_call(
        _csq_body,
        grid=(B, N // RT),
        in_specs=[
            pl.BlockSpec((1, RT, W), lambda b, i: (b, i, 0)),
            pl.BlockSpec((8, W), lambda b, i: (0, 0)),
        ],
        out_specs=pl.BlockSpec((8, W), lambda b, i: (0, 0)),
        out_shape=jax.ShapeDtypeStruct((8, W), F32),
    )(y, s)


def _g_body(m5_ref, s5_ref, q5_ref, w6b_ref, g5b5_ref, v6_ref):
    nt = (((1,), (1,)), ((), ()))
    cnt2 = float(M)
    mean5 = s5_ref[0:1, :] / cnt2
    var5 = q5_ref[0:1, :] / cnt2
    xh5 = (m5_ref[...] - mean5) / jnp.sqrt(var5 + EPS)
    gvec = jnp.maximum(xh5 * g5b5_ref[0:1] + g5b5_ref[1:2], 0.0)
    v6_ref[...] = lax.dot_general(gvec, w6b_ref[...], nt, preferred_element_type=F32)


def _gstage(m5, s5, q5, W6b, g5b5):
    return pl.pallas_call(
        _g_body,
        in_specs=[pl.BlockSpec(a.shape, None) for a in (m5, s5, q5, W6b, g5b5)],
        out_specs=pl.BlockSpec((B, 512), None),
        out_shape=jax.ShapeDtypeStruct((B, 512), F32),
    )(m5, s5, q5, W6b, g5b5)


def _k1_body(x_ref, w_ref, v6_ref, y_ref, st_ref):
    b = pl.program_id(0)
    i = pl.program_id(1)
    nt = (((1,), (1,)), ((), ()))
    y = (lax.dot_general(x_ref[0], w_ref[...], nt, preferred_element_type=F32)
         + v6_ref[pl.ds(b, 1), :])
    y_ref[0] = y

    @pl.when(jnp.logical_and(b == 0, i == 0))
    def _():
        st_ref[...] = jnp.zeros_like(st_ref)

    st_ref[0:1, :] += jnp.sum(y, axis=0, keepdims=True)


def _k1(x1, W6a, v6):
    return pl.pallas_call(
        _k1_body,
        grid=(B, N // RT),
        in_specs=[
            pl.BlockSpec((1, RT, 64), lambda b, i: (b, i, 0)),
            pl.BlockSpec((512, 64), lambda b, i: (0, 0)),
            pl.BlockSpec((B, 512), lambda b, i: (0, 0)),
        ],
        out_specs=[
            pl.BlockSpec((1, RT, 512), lambda b, i: (b, i, 0)),
            pl.BlockSpec((8, 512), lambda b, i: (0, 0)),
        ],
        out_shape=[
            jax.ShapeDtypeStruct((B, N, 512), F32),
            jax.ShapeDtypeStruct((8, 512), F32),
        ],
    )(x1, W6a, v6)


def _k2_body(y_ref, st_ref, q_ref, gb_ref, w_ref, o_ref, st2_ref):
    b = pl.program_id(0)
    i = pl.program_id(1)
    cnt2 = float(M)
    mean = st_ref[0:1, :] / cnt2
    var = q_ref[0:1, :] / cnt2
    xh = (y_ref[0] - mean) / jnp.sqrt(var + EPS)
    h = jnp.maximum(xh * gb_ref[0:1] + gb_ref[1:2], 0.0)
    nt = (((1,), (1,)), ((), ()))
    y2 = lax.dot_general(h, w_ref[...], nt, preferred_element_type=F32)
    o_ref[0] = y2

    @pl.when(jnp.logical_and(b == 0, i == 0))
    def _():
        st2_ref[...] = jnp.zeros_like(st2_ref)

    st2_ref[0:1, :] += jnp.sum(y2, axis=0, keepdims=True)


def _k2(y6, st6, q6, g6b6, W7):
    return pl.pallas_call(
        _k2_body,
        grid=(B, N // RT),
        in_specs=[
            pl.BlockSpec((1, RT, 512), lambda b, i: (b, i, 0)),
            pl.BlockSpec((8, 512), lambda b, i: (0, 0)),
            pl.BlockSpec((8, 512), lambda b, i: (0, 0)),
            pl.BlockSpec((8, 512), lambda b, i: (0, 0)),
            pl.BlockSpec((128, 512), lambda b, i: (0, 0)),
        ],
        out_specs=[
            pl.BlockSpec((1, RT, 128), lambda b, i: (b, i, 0)),
            pl.BlockSpec((8, 128), lambda b, i: (0, 0)),
        ],
        out_shape=[
            jax.ShapeDtypeStruct((B, N, 128), F32),
            jax.ShapeDtypeStruct((8, 128), F32),
        ],
    )(y6, st6, q6, g6b6, W7)


def _k3_body(y_ref, st_ref, q_ref, gb_ref, w_ref, o_ref):
    cnt2 = float(M)
    mean = st_ref[0:1, :] / cnt2
    var = q_ref[0:1, :] / cnt2
    xh = (y_ref[0] - mean) / jnp.sqrt(var + EPS)
    h = jnp.maximum(xh * gb_ref[0:1] + gb_ref[1:2], 0.0)
    nt = (((1,), (1,)), ((), ()))
    o_ref[0] = jnp.maximum(
        lax.dot_general(h, w_ref[...], nt, preferred_element_type=F32), 0.0)


def _k3(y7, st7, q7, g7b7, W8p):
    return pl.pallas_call(
        _k3_body,
        grid=(B, N // RT),
        in_specs=[
            pl.BlockSpec((1, RT, 128), lambda b, i: (b, i, 0)),
            pl.BlockSpec((8, 128), lambda b, i: (0, 0)),
            pl.BlockSpec((8, 128), lambda b, i: (0, 0)),
            pl.BlockSpec((8, 128), lambda b, i: (0, 0)),
            pl.BlockSpec((128, 128), lambda b, i: (0, 0)),
        ],
        out_specs=pl.BlockSpec((1, RT, 128), lambda b, i: (b, i, 0)),
        out_shape=jax.ShapeDtypeStruct((B, N, 128), F32),
    )(y7, st7, q7, g7b7, W8p)


def _gbpack(g, b, width):
    z = jnp.zeros((8, width), F32)
    return z.at[0].set(g).at[1].set(b)


def kernel(x, params):
    p = params
    xT = jnp.transpose(x, (0, 2, 1))
    xTp = jnp.pad(xT, ((0, 0), (0, 0), (0, 5)))
    W1p = jnp.pad(p['W1'], ((0, 0), (0, 5)))

    h = _edge_layer(xTp, W1p, _gbpack(p['g1'], p['b1'], 64))
    x1 = _edge_layer(h, p['W2'], _gbpack(p['g2'], p['b2'], 64))
    h = _edge_layer(x1, p['W3'], _gbpack(p['g3'], p['b3'], 64))
    h4 = _edge_layer(h, p['W4'], _gbpack(p['g4'], p['b4'], 64))

    y5, s5, m5 = _m1(h4, p['W5'])
    q5 = _csq(y5, s5)
    W6a = p['W6'][:, :64]
    W6b = p['W6'][:, 64:]
    v6 = _gstage(m5, s5, q5, W6b, _gbpack(p['g5'], p['b5'], 1024))
    y6, st6 = _k1(x1, W6a, v6)
    q6 = _csq(y6, st6)
    y7, st7 = _k2(y6, st6, q6, _gbpack(p['g6'], p['b6'], 512), p['W7'])
    q7 = _csq(y7, st7)
    W8p = jnp.pad(p['W8'], ((0, 115), (0, 0)))
    outp = _k3(y7, st7, q7, _gbpack(p['g7'], p['b7'], 128), W8p)
    return jnp.transpose(outp[:, :, :13], (0, 2, 1))
